# baseline (device time: 85828 ns/iter reference)
import jax
import jax.numpy as jnp
from jax import lax
from jax.experimental import pallas as pl
from jax.experimental.pallas import tpu as pltpu

N_DEV = 4
N_TOK = 1024
D_IN = 256
D_OUT = 512
N_EXP_LOCAL = 4
CAPACITY = 51


def kernel(x, router_W, route_idx, expert_W):
    def body(x_ref, rw_ref, ridx_ref, ew_ref, out_ref,
             comm_ref, send_sems, recv_sems):
        my = lax.axis_index("i")
        left = (my + N_DEV - 1) % N_DEV
        right = (my + 1) % N_DEV

        ridx = ridx_ref[:, :]
        e_iota = lax.broadcasted_iota(jnp.int32, (1, 16), 1)
        onehot = (ridx == e_iota).astype(jnp.float32)
        row = lax.broadcasted_iota(jnp.int32, (N_TOK, N_TOK), 0)
        col = lax.broadcasted_iota(jnp.int32, (N_TOK, N_TOK), 1)
        tril = (col < row).astype(jnp.float32)
        cum = jnp.dot(tril, onehot, preferred_element_type=jnp.float32)
        pos = jnp.sum(cum * onehot, axis=1, keepdims=True)
        keep = pos < float(CAPACITY)

        xv = x_ref[:, :]
        acc = jnp.zeros((N_TOK, D_OUT), jnp.float32)
        for le in range(N_EXP_LOCAL):
            eg = my * N_EXP_LOCAL + le
            m = jnp.where((ridx == eg) & keep, 1.0, 0.0)
            acc = acc + jnp.dot(xv * m, ew_ref[le],
                                preferred_element_type=jnp.float32)
        out_ref[:, :] = acc
        comm_ref[0, :, :] = acc

        barrier_sem = pltpu.get_barrier_semaphore()
        for nbr in (left, right):
            pl.semaphore_signal(barrier_sem, inc=1, device_id=(nbr,),
                                device_id_type=pl.DeviceIdType.MESH)
        pl.semaphore_wait(barrier_sem, 2)

        for h in range(N_DEV - 1):
            rdma = pltpu.make_async_remote_copy(
                src_ref=comm_ref.at[h],
                dst_ref=comm_ref.at[h + 1],
                send_sem=send_sems.at[h],
                recv_sem=recv_sems.at[h],
                device_id=(right,),
                device_id_type=pl.DeviceIdType.MESH,
            )
            rdma.start()
            rdma.wait()
            out_ref[:, :] += comm_ref[h + 1, :, :]

    return pl.pallas_call(
        body,
        out_shape=jax.ShapeDtypeStruct((N_TOK, D_OUT), jnp.float32),
        in_specs=[
            pl.BlockSpec(memory_space=pltpu.VMEM),
            pl.BlockSpec(memory_space=pltpu.VMEM),
            pl.BlockSpec(memory_space=pltpu.VMEM),
            pl.BlockSpec(memory_space=pltpu.VMEM),
        ],
        out_specs=pl.BlockSpec(memory_space=pltpu.VMEM),
        scratch_shapes=[
            pltpu.VMEM((N_DEV, N_TOK, D_OUT), jnp.float32),
            pltpu.SemaphoreType.DMA((N_DEV - 1,)),
            pltpu.SemaphoreType.DMA((N_DEV - 1,)),
        ],
        compiler_params=pltpu.CompilerParams(collective_id=0),
    )(x, router_W, route_idx, expert_W)


# device time: 34903 ns/iter; 2.4590x vs baseline; 2.4590x over previous
import jax
import jax.numpy as jnp
from jax import lax
from jax.experimental import pallas as pl
from jax.experimental.pallas import tpu as pltpu

N_DEV = 4
N_TOK = 1024
D_IN = 256
D_OUT = 512
N_EXP_LOCAL = 4
CAPACITY = 51
SLOT = 64
C = N_EXP_LOCAL * SLOT


def kernel(x, router_W, route_idx, expert_W):
    def body(x_ref, rw_ref, ridx_ref, ew_ref, out_ref,
             comm_ref, send_sems, recv_sems):
        my = lax.axis_index("i")
        left = (my + N_DEV - 1) % N_DEV
        right = (my + 1) % N_DEV

        ridx = ridx_ref[:, :]
        e_iota = lax.broadcasted_iota(jnp.int32, (1, 16), 1)
        onehot = (ridx == e_iota).astype(jnp.float32)
        row = lax.broadcasted_iota(jnp.int32, (N_TOK, N_TOK), 0)
        col = lax.broadcasted_iota(jnp.int32, (N_TOK, N_TOK), 1)
        tril = (col < row).astype(jnp.float32)
        cum = jnp.dot(tril, onehot, preferred_element_type=jnp.float32)
        pos = jnp.sum(cum * onehot, axis=1, keepdims=True)
        pos_i = pos.astype(jnp.int32)
        keep = pos_i < CAPACITY

        c_iota = lax.broadcasted_iota(jnp.int32, (1, C), 1)
        c_exp = c_iota // SLOT
        c_slot = c_iota % SLOT

        def selection(origin):
            e_c = origin * N_EXP_LOCAL + c_exp
            return ((ridx == e_c) & (pos_i == c_slot) & keep).astype(
                jnp.float32)

        s_own = selection(my)

        compact_x = lax.dot_general(
            s_own, x_ref[:, :], (((0,), (0,)), ((), ())),
            preferred_element_type=jnp.float32)
        for le in range(N_EXP_LOCAL):
            comm_ref[0, le * SLOT:(le + 1) * SLOT, :] = jnp.dot(
                compact_x[le * SLOT:(le + 1) * SLOT, :], ew_ref[le],
                preferred_element_type=jnp.float32)

        out_ref[:, :] = jnp.dot(s_own, comm_ref[0, :, :],
                                preferred_element_type=jnp.float32)

        barrier_sem = pltpu.get_barrier_semaphore()
        for nbr in (left, right):
            pl.semaphore_signal(barrier_sem, inc=1, device_id=(nbr,),
                                device_id_type=pl.DeviceIdType.MESH)
        pl.semaphore_wait(barrier_sem, 2)

        for h in range(N_DEV - 1):
            rdma = pltpu.make_async_remote_copy(
                src_ref=comm_ref.at[h],
                dst_ref=comm_ref.at[h + 1],
                send_sem=send_sems.at[h],
                recv_sem=recv_sems.at[h],
                device_id=(right,),
                device_id_type=pl.DeviceIdType.MESH,
            )
            rdma.start()
            rdma.wait()
            origin = (my + N_DEV - 1 - h) % N_DEV
            out_ref[:, :] += jnp.dot(selection(origin), comm_ref[h + 1, :, :],
                                     preferred_element_type=jnp.float32)

    return pl.pallas_call(
        body,
        out_shape=jax.ShapeDtypeStruct((N_TOK, D_OUT), jnp.float32),
        in_specs=[
            pl.BlockSpec(memory_space=pltpu.VMEM),
            pl.BlockSpec(memory_space=pltpu.VMEM),
            pl.BlockSpec(memory_space=pltpu.VMEM),
            pl.BlockSpec(memory_space=pltpu.VMEM),
        ],
        out_specs=pl.BlockSpec(memory_space=pltpu.VMEM),
        scratch_shapes=[
            pltpu.VMEM((N_DEV, C, D_OUT), jnp.float32),
            pltpu.SemaphoreType.DMA((N_DEV - 1,)),
            pltpu.SemaphoreType.DMA((N_DEV - 1,)),
        ],
        compiler_params=pltpu.CompilerParams(collective_id=0),
    )(x, router_W, route_idx, expert_W)


# device time: 20693 ns/iter; 4.1477x vs baseline; 1.6867x over previous
import jax
import jax.numpy as jnp
from jax import lax
from jax.experimental import pallas as pl
from jax.experimental.pallas import tpu as pltpu

N_DEV = 4
N_TOK = 1024
D_IN = 256
D_OUT = 512
N_EXP_LOCAL = 4
CAPACITY = 51
SLOT = 64
C = N_EXP_LOCAL * SLOT

OWN, FROM_L, FROM_R, DIAG = 0, 1, 2, 3
S_L, S_R, S_F, R_FROM_L, R_FROM_R, R_DIAG = 0, 1, 2, 3, 4, 5


def kernel(x, router_W, route_idx, expert_W):
    def body(x_ref, rw_ref, ridx_ref, ew_ref, out_ref, comm_ref, sems):
        my = lax.axis_index("i")
        left = (my + N_DEV - 1) % N_DEV
        right = (my + 1) % N_DEV

        barrier_sem = pltpu.get_barrier_semaphore()
        for nbr in (left, right):
            pl.semaphore_signal(barrier_sem, inc=1, device_id=(nbr,),
                                device_id_type=pl.DeviceIdType.MESH)
        pl.semaphore_wait(barrier_sem, 2)

        ridx = ridx_ref[:, :]
        e_iota = lax.broadcasted_iota(jnp.int32, (1, 16), 1)
        onehot = (ridx == e_iota).astype(jnp.float32)
        row = lax.broadcasted_iota(jnp.int32, (N_TOK, N_TOK), 0)
        col = lax.broadcasted_iota(jnp.int32, (N_TOK, N_TOK), 1)
        tril = (col < row).astype(jnp.float32)
        cum = jnp.dot(tril, onehot, preferred_element_type=jnp.float32)
        pos = jnp.sum(cum * onehot, axis=1, keepdims=True)
        pos_i = pos.astype(jnp.int32)
        keep = pos_i < CAPACITY

        c_iota = lax.broadcasted_iota(jnp.int32, (1, C), 1)
        c_exp = c_iota // SLOT
        c_slot = c_iota % SLOT

        def selection(origin):
            e_c = origin * N_EXP_LOCAL + c_exp
            return ((ridx == e_c) & (pos_i == c_slot) & keep).astype(
                jnp.bfloat16)

        s_own = selection(my)

        xb = x_ref[:, :].astype(jnp.bfloat16)
        compact_x = lax.dot_general(
            s_own, xb, (((0,), (0,)), ((), ())),
            preferred_element_type=jnp.float32)
        cxb = compact_x.astype(jnp.bfloat16)
        for le in range(N_EXP_LOCAL):
            blk = jnp.dot(cxb[le * SLOT:(le + 1) * SLOT, :],
                          ew_ref[le].astype(jnp.bfloat16),
                          preferred_element_type=jnp.float32)
            comm_ref[OWN, le * SLOT:(le + 1) * SLOT, :] = blk.astype(
                jnp.bfloat16)

        send_l = pltpu.make_async_remote_copy(
            src_ref=comm_ref.at[OWN], dst_ref=comm_ref.at[FROM_R],
            send_sem=sems.at[S_L], recv_sem=sems.at[R_FROM_R],
            device_id=(left,), device_id_type=pl.DeviceIdType.MESH)
        send_r = pltpu.make_async_remote_copy(
            src_ref=comm_ref.at[OWN], dst_ref=comm_ref.at[FROM_L],
            send_sem=sems.at[S_R], recv_sem=sems.at[R_FROM_L],
            device_id=(right,), device_id_type=pl.DeviceIdType.MESH)
        fwd = pltpu.make_async_remote_copy(
            src_ref=comm_ref.at[FROM_R], dst_ref=comm_ref.at[DIAG],
            send_sem=sems.at[S_F], recv_sem=sems.at[R_DIAG],
            device_id=(left,), device_id_type=pl.DeviceIdType.MESH)

        send_l.start()
        send_r.start()

        out = jnp.dot(s_own, comm_ref[OWN, :, :],
                      preferred_element_type=jnp.float32)

        send_l.wait_recv()
        fwd.start()
        out += jnp.dot(selection(right), comm_ref[FROM_R, :, :],
                       preferred_element_type=jnp.float32)

        send_r.wait_recv()
        out += jnp.dot(selection(left), comm_ref[FROM_L, :, :],
                       preferred_element_type=jnp.float32)

        fwd.wait_recv()
        out += jnp.dot(selection((my + 2) % N_DEV), comm_ref[DIAG, :, :],
                       preferred_element_type=jnp.float32)

        out_ref[:, :] = out

        send_l.wait_send()
        send_r.wait_send()
        fwd.wait_send()

    return pl.pallas_call(
        body,
        out_shape=jax.ShapeDtypeStruct((N_TOK, D_OUT), jnp.float32),
        in_specs=[
            pl.BlockSpec(memory_space=pltpu.VMEM),
            pl.BlockSpec(memory_space=pltpu.VMEM),
            pl.BlockSpec(memory_space=pltpu.VMEM),
            pl.BlockSpec(memory_space=pltpu.VMEM),
        ],
        out_specs=pl.BlockSpec(memory_space=pltpu.VMEM),
        scratch_shapes=[
            pltpu.VMEM((4, C, D_OUT), jnp.bfloat16),
            pltpu.SemaphoreType.DMA((6,)),
        ],
        compiler_params=pltpu.CompilerParams(collective_id=0),
    )(x, router_W, route_idx, expert_W)


# device time: 18446 ns/iter; 4.6529x vs baseline; 1.1218x over previous
import jax
import jax.numpy as jnp
from jax import lax
from jax.experimental import pallas as pl
from jax.experimental.pallas import tpu as pltpu

N_DEV = 4
N_TOK = 1024
D_IN = 256
D_OUT = 512
N_EXP_LOCAL = 4
CAPACITY = 51
SLOT = 56
C = N_EXP_LOCAL * SLOT

OWN, FROM_L, FROM_R, DIAG = 0, 1, 2, 3
SL0, SL1, SR0, SR1, SFL, SFR, RL0, RL1, RR0, RR1, RDT, RDB = range(12)
HALF = C // 2


def kernel(x, router_W, route_idx, expert_W):
    def body(x_ref, rw_ref, ridx_ref, ew_ref, out_ref, comm_ref, sems):
        my = lax.axis_index("i")
        left = (my + N_DEV - 1) % N_DEV
        right = (my + 1) % N_DEV

        barrier_sem = pltpu.get_barrier_semaphore()
        for nbr in (left, right):
            pl.semaphore_signal(barrier_sem, inc=1, device_id=(nbr,),
                                device_id_type=pl.DeviceIdType.MESH)
        pl.semaphore_wait(barrier_sem, 2)

        ridx = ridx_ref[:, :]
        e_iota = lax.broadcasted_iota(jnp.int32, (1, 16), 1)
        onehot = (ridx == e_iota).astype(jnp.bfloat16)
        row = lax.broadcasted_iota(jnp.int32, (N_TOK, N_TOK), 0)
        col = lax.broadcasted_iota(jnp.int32, (N_TOK, N_TOK), 1)
        tril = (col < row).astype(jnp.bfloat16)
        cum = jnp.dot(tril, onehot, preferred_element_type=jnp.float32)
        pos = jnp.sum(cum * onehot.astype(jnp.float32), axis=1,
                      keepdims=True)
        pos_i = pos.astype(jnp.int32)
        keep = pos_i < CAPACITY

        c_iota = lax.broadcasted_iota(jnp.int32, (1, C), 1)
        c_exp = c_iota // SLOT
        c_slot = c_iota % SLOT

        def selection(origin):
            e_c = origin * N_EXP_LOCAL + c_exp
            return ((ridx == e_c) & (pos_i == c_slot) & keep).astype(
                jnp.bfloat16)

        s_own = selection(my)

        xb = x_ref[:, :].astype(jnp.bfloat16)
        compact_x = lax.dot_general(
            s_own, xb, (((0,), (0,)), ((), ())),
            preferred_element_type=jnp.float32)
        cxb = compact_x.astype(jnp.bfloat16)

        def hop1(lo, hi, s_l, r_r, s_r, r_l):
            snd_l = pltpu.make_async_remote_copy(
                src_ref=comm_ref.at[OWN, lo:hi],
                dst_ref=comm_ref.at[FROM_R, lo:hi],
                send_sem=sems.at[s_l], recv_sem=sems.at[r_r],
                device_id=(left,), device_id_type=pl.DeviceIdType.MESH)
            snd_r = pltpu.make_async_remote_copy(
                src_ref=comm_ref.at[OWN, lo:hi],
                dst_ref=comm_ref.at[FROM_L, lo:hi],
                send_sem=sems.at[s_r], recv_sem=sems.at[r_l],
                device_id=(right,), device_id_type=pl.DeviceIdType.MESH)
            return snd_l, snd_r

        fwd_l = pltpu.make_async_remote_copy(
            src_ref=comm_ref.at[FROM_R, 0:HALF],
            dst_ref=comm_ref.at[DIAG, 0:HALF],
            send_sem=sems.at[SFL], recv_sem=sems.at[RDT],
            device_id=(left,), device_id_type=pl.DeviceIdType.MESH)
        fwd_r = pltpu.make_async_remote_copy(
            src_ref=comm_ref.at[FROM_L, HALF:C],
            dst_ref=comm_ref.at[DIAG, HALF:C],
            send_sem=sems.at[SFR], recv_sem=sems.at[RDB],
            device_id=(right,), device_id_type=pl.DeviceIdType.MESH)

        for le in range(2):
            blk = jnp.dot(cxb[le * SLOT:(le + 1) * SLOT, :],
                          ew_ref[le].astype(jnp.bfloat16),
                          preferred_element_type=jnp.float32)
            comm_ref[OWN, le * SLOT:(le + 1) * SLOT, :] = blk.astype(
                jnp.bfloat16)
        send_l0, send_r0 = hop1(0, HALF, SL0, RR0, SR0, RL0)
        send_l0.start()
        send_r0.start()

        for le in range(2, N_EXP_LOCAL):
            blk = jnp.dot(cxb[le * SLOT:(le + 1) * SLOT, :],
                          ew_ref[le].astype(jnp.bfloat16),
                          preferred_element_type=jnp.float32)
            comm_ref[OWN, le * SLOT:(le + 1) * SLOT, :] = blk.astype(
                jnp.bfloat16)
        send_l1, send_r1 = hop1(HALF, C, SL1, RR1, SR1, RL1)
        send_l1.start()
        send_r1.start()

        out = jnp.dot(s_own, comm_ref[OWN, :, :],
                      preferred_element_type=jnp.float32)

        send_l0.wait_recv()
        fwd_l.start()
        send_r1.wait_recv()
        fwd_r.start()

        send_l1.wait_recv()
        out += jnp.dot(selection(right), comm_ref[FROM_R, :, :],
                       preferred_element_type=jnp.float32)

        send_r0.wait_recv()
        out += jnp.dot(selection(left), comm_ref[FROM_L, :, :],
                       preferred_element_type=jnp.float32)

        fwd_l.wait_recv()
        fwd_r.wait_recv()
        out += jnp.dot(selection((my + 2) % N_DEV), comm_ref[DIAG, :, :],
                       preferred_element_type=jnp.float32)

        out_ref[:, :] = out

        for d in (send_l0, send_r0, send_l1, send_r1, fwd_l, fwd_r):
            d.wait_send()

    return pl.pallas_call(
        body,
        out_shape=jax.ShapeDtypeStruct((N_TOK, D_OUT), jnp.float32),
        in_specs=[
            pl.BlockSpec(memory_space=pltpu.VMEM),
            pl.BlockSpec(memory_space=pltpu.VMEM),
            pl.BlockSpec(memory_space=pltpu.VMEM),
            pl.BlockSpec(memory_space=pltpu.VMEM),
        ],
        out_specs=pl.BlockSpec(memory_space=pltpu.VMEM),
        scratch_shapes=[
            pltpu.VMEM((4, C, D_OUT), jnp.bfloat16),
            pltpu.SemaphoreType.DMA((12,)),
        ],
        compiler_params=pltpu.CompilerParams(collective_id=0),
    )(x, router_W, route_idx, expert_W)
